# baseline (speedup 1.0000x reference)
import functools

import jax
import jax.numpy as jnp
from jax.experimental import pallas as pl

K = 8
B = 4.0
SOFTMAX_ADJUST = 0.01
MIN_DERIVATIVE = 0.001
PAD_LO = -40000.0
PAD_HI = 40000.0


def _spline_body(x_ref, p_ref, o_ref):
    x = x_ref[0, 0]
    p = p_ref[0]
    E = x.shape[0]
    f32 = jnp.float32

    lane12 = jax.lax.broadcasted_iota(jnp.int32, (E, 12), 1)

    def knot_positions(q):
        m = jnp.max(q, axis=-1, keepdims=True)
        w = jnp.exp(q - m)
        s = jnp.sum(w, axis=-1, keepdims=True)
        w = w / s
        w = (w + SOFTMAX_ADJUST / K) / (1.0 + SOFTMAX_ADJUST)
        lane = jax.lax.broadcasted_iota(jnp.int32, (E, K), 1)
        w = jnp.where(lane == 0, 0.5 * w, w)
        i = jax.lax.broadcasted_iota(jnp.int32, (K, K), 0)
        j = jax.lax.broadcasted_iota(jnp.int32, (K, K), 1)
        tri = (i <= j).astype(f32)
        cs = jax.lax.dot_general(w, tri, (((1,), (0,)), ((), ())),
                                 preferred_element_type=f32,
                                 precision=jax.lax.Precision.HIGHEST)
        return 2.0 * B * cs - B

    xc = knot_positions(p[:, 0:K])
    yc = knot_positions(p[:, K:2 * K])

    t = p[:, 2 * K:2 * K + (K - 1)]
    d = jnp.maximum(t, 0.0) + jnp.log1p(jnp.exp(-jnp.abs(t))) + MIN_DERIVATIVE

    pad_lo2 = jnp.full((E, 1), PAD_LO, f32)
    pad_lo1 = jnp.full((E, 1), -B, f32)
    pad_hi1 = jnp.full((E, 1), B, f32)
    pad_hi2 = jnp.full((E, 1), PAD_HI, f32)
    ones1 = jnp.ones((E, 1), f32)

    xpos = jnp.concatenate([pad_lo2, pad_lo1, xc, pad_hi1, pad_hi2], axis=-1)
    ypos = jnp.concatenate([pad_lo2, pad_lo1, yc, pad_hi1, pad_hi2], axis=-1)
    dpad = jnp.concatenate([ones1, ones1, d, ones1, ones1], axis=-1)

    xb = x[:, None]
    cnt = jnp.sum((xpos < xb).astype(jnp.int32), axis=-1)
    k = cnt - 1
    kb = k[:, None]

    oh0 = (lane12 == kb).astype(f32)
    oh1 = (lane12 == kb + 1).astype(f32)

    xk = jnp.sum(xpos * oh0, axis=-1)
    xk1 = jnp.sum(xpos * oh1, axis=-1)
    yk = jnp.sum(ypos * oh0, axis=-1)
    yk1 = jnp.sum(ypos * oh1, axis=-1)
    lane11 = lane12[:, :11]
    dk = jnp.sum(dpad * (lane11 == kb).astype(f32), axis=-1)
    dk1 = jnp.sum(dpad * (lane11 == jnp.minimum(kb + 1, 10)).astype(f32),
                  axis=-1)

    dx = xk1 - xk
    dy = yk1 - yk
    xi = (x - xk) / dx
    sk = dy / dx
    omxi = 1.0 - xi
    xiomxi = xi * omxi
    num = dy * (sk * xi * xi + dk * xiomxi)
    den = sk + (dk1 + dk - 2.0 * sk) * xiomxi
    o_ref[0, 0] = yk + num / den


@jax.jit
def kernel(x, params):
    dim = x.shape[0]
    e = 4096
    g = dim // e
    xg = x.reshape(g, 1, e)
    pg = params.reshape(g, e, 23)
    out = pl.pallas_call(
        _spline_body,
        grid=(g,),
        in_specs=[
            pl.BlockSpec((1, 1, e), lambda i: (i, 0, 0)),
            pl.BlockSpec((1, e, 23), lambda i: (i, 0, 0)),
        ],
        out_specs=pl.BlockSpec((1, 1, e), lambda i: (i, 0, 0)),
        out_shape=jax.ShapeDtypeStruct((g, 1, e), jnp.float32),
    )(xg, pg)
    return out.reshape(dim)


# speedup vs baseline: 5.6588x; 5.6588x over previous
import jax
import jax.numpy as jnp
from jax import lax
from jax.experimental import pallas as pl
from jax.experimental.pallas import tpu as pltpu
from jax.experimental.pallas import tpu_sc as plsc

K = 8
B = 4.0
ADJ = 0.01
MIN_DERIVATIVE = 0.001
PAD_LO = -40000.0
PAD_HI = 40000.0

NC = 2
NS = 16
L = 16
NW = NC * NS

CH = 2048
GRP = CH // L

A_SC = 2.0 * B / (1.0 + ADJ)
BETA = 2.0 * B * (ADJ / K) / (1.0 + ADJ)


def _softplus(t):
    e = jnp.exp(-jnp.abs(t))
    z = e / (e + 2.0)
    z2 = z * z
    p = 2.0 / 9.0 + z2 * (2.0 / 11.0)
    p = 2.0 / 7.0 + z2 * p
    p = 2.0 / 5.0 + z2 * p
    p = 2.0 / 3.0 + z2 * p
    p = 2.0 + z2 * p
    return jnp.maximum(t, 0.0) + z * p


def _sc_body(x_hbm, p_hbm, y_hbm, xv, pv, yv, posv):
    wid = lax.axis_index("s") * NC + lax.axis_index("c")
    per_w = x_hbm.shape[0] // NW
    nch = per_w // CH
    lane = lax.iota(jnp.int32, L)
    lane23 = lane * 23
    ones = jnp.ones((L,), jnp.float32)

    posv[pl.ds(0 * L, L)] = PAD_LO * ones
    posv[pl.ds(1 * L, L)] = -B * ones
    posv[pl.ds(10 * L, L)] = B * ones
    posv[pl.ds(11 * L, L)] = PAD_HI * ones
    posv[pl.ds(12 * L, L)] = PAD_LO * ones
    posv[pl.ds(13 * L, L)] = -B * ones
    posv[pl.ds(22 * L, L)] = B * ones
    posv[pl.ds(23 * L, L)] = PAD_HI * ones

    def chunk_body(ci, carry):
        base = wid * per_w + ci * CH
        pltpu.sync_copy(x_hbm.at[pl.ds(base, CH)], xv)
        pltpu.sync_copy(p_hbm.at[pl.ds(base * 23, CH * 23)], pv)

        def grp(g, carry2):
            rows23 = g * (L * 23) + lane23
            x = xv[pl.ds(g * L, L)]

            def knots(off, out_row):
                w0 = jnp.exp(plsc.load_gather(pv, [rows23 + off]))
                half_w0 = 0.5 * w0
                cs = [w0]
                for j in range(1, K):
                    cs.append(cs[-1] + jnp.exp(
                        plsc.load_gather(pv, [rows23 + (off + j)])))
                scale = A_SC / cs[-1]
                pos = []
                for j in range(K):
                    pj = (cs[j] - half_w0) * scale + (BETA * (j + 0.5) - B)
                    posv[pl.ds((out_row + j) * L, L)] = pj
                    pos.append(pj)
                return pos

            xp = knots(0, 2)
            knots(K, 14)

            cnt = jnp.where(x > -B, 2, 1)
            for j in range(K):
                cnt = cnt + (xp[j] < x).astype(jnp.int32)
            cnt = cnt + (x > B).astype(jnp.int32)
            k = cnt - 1

            idx = k * L + lane
            xk = plsc.load_gather(posv, [idx])
            xk1 = plsc.load_gather(posv, [idx + L])
            yk = plsc.load_gather(posv, [idx + 12 * L])
            yk1 = plsc.load_gather(posv, [idx + 13 * L])

            jd = jnp.clip(k - 2, 0, 6) + 2 * K
            jd1 = jnp.clip(k - 1, 0, 6) + 2 * K
            pdk = plsc.load_gather(pv, [rows23 + jd])
            pdk1 = plsc.load_gather(pv, [rows23 + jd1])
            interior = jnp.logical_and(k > 1, k < 9)
            interior1 = jnp.logical_and(k > 0, k < 8)
            dk = jnp.where(interior, _softplus(pdk) + MIN_DERIVATIVE, 1.0)
            dk1 = jnp.where(interior1, _softplus(pdk1) + MIN_DERIVATIVE, 1.0)

            dx = xk1 - xk
            dy = yk1 - yk
            xi = (x - xk) / dx
            sk = dy / dx
            xo = xi * (1.0 - xi)
            num = dy * (sk * xi * xi + dk * xo)
            den = sk + (dk1 + dk - 2.0 * sk) * xo
            yv[pl.ds(g * L, L)] = yk + num / den
            return carry2

        lax.fori_loop(0, GRP, grp, 0)
        pltpu.sync_copy(yv, y_hbm.at[pl.ds(base, CH)])
        return carry

    lax.fori_loop(0, nch, chunk_body, 0)


@jax.jit
def kernel(x, params):
    f = pl.kernel(
        _sc_body,
        out_type=jax.ShapeDtypeStruct(x.shape, jnp.float32),
        mesh=plsc.VectorSubcoreMesh(core_axis_name="c", subcore_axis_name="s"),
        compiler_params=pltpu.CompilerParams(needs_layout_passes=False),
        scratch_types=[
            pltpu.VMEM((CH,), jnp.float32),
            pltpu.VMEM((CH * 23,), jnp.float32),
            pltpu.VMEM((CH,), jnp.float32),
            pltpu.VMEM((24 * L,), jnp.float32),
        ],
    )
    return f(x, params.reshape(-1))


# speedup vs baseline: 5.8648x; 1.0364x over previous
import jax
import jax.numpy as jnp
from jax import lax
from jax.experimental import pallas as pl
from jax.experimental.pallas import tpu as pltpu
from jax.experimental.pallas import tpu_sc as plsc

K = 8
B = 4.0
ADJ = 0.01
MIN_DERIVATIVE = 0.001
PAD_LO = -40000.0
PAD_HI = 40000.0

NC = 2
NS = 16
L = 16
NW = NC * NS

CH = 2048
GRP = CH // L
TBL = 24 * L

A_SC = 2.0 * B / (1.0 + ADJ)
BETA = 2.0 * B * (ADJ / K) / (1.0 + ADJ)


def _softplus2(t1, t2):
    e1 = jnp.exp(-jnp.abs(t1))
    e2 = jnp.exp(-jnp.abs(t2))
    a = e1 + 2.0
    b = e2 + 2.0
    r = 1.0 / (a * b)
    outs = []
    for t, e, other in ((t1, e1, b), (t2, e2, a)):
        z = e * other * r
        z2 = z * z
        p = 2.0 / 9.0 + z2 * (2.0 / 11.0)
        p = 2.0 / 7.0 + z2 * p
        p = 2.0 / 5.0 + z2 * p
        p = 2.0 / 3.0 + z2 * p
        p = 2.0 + z2 * p
        outs.append(jnp.maximum(t, 0.0) + z * p)
    return outs


def _prefix8(w):
    s01 = w[0] + w[1]
    s23 = w[2] + w[3]
    s45 = w[4] + w[5]
    s67 = w[6] + w[7]
    s03 = s01 + s23
    s47 = s45 + s67
    total = s03 + s47
    c = [w[0], s01, s01 + w[2], s03, s03 + w[4], s03 + s45,
         s03 + s45 + w[6], total]
    return c


def _sc_body(x_hbm, p_hbm, y_hbm, xv, pv, yv, posv):
    wid = lax.axis_index("s") * NC + lax.axis_index("c")
    per_w = x_hbm.shape[0] // NW
    nch = per_w // CH
    lane = lax.iota(jnp.int32, L)
    lane23 = lane * 23
    ones = jnp.ones((L,), jnp.float32)

    for r in range(2):
        o = r * TBL
        posv[pl.ds(o + 0 * L, L)] = PAD_LO * ones
        posv[pl.ds(o + 1 * L, L)] = -B * ones
        posv[pl.ds(o + 10 * L, L)] = B * ones
        posv[pl.ds(o + 11 * L, L)] = PAD_HI * ones
        posv[pl.ds(o + 12 * L, L)] = PAD_LO * ones
        posv[pl.ds(o + 13 * L, L)] = -B * ones
        posv[pl.ds(o + 22 * L, L)] = B * ones
        posv[pl.ds(o + 23 * L, L)] = PAD_HI * ones

    def group(g, tbl):
        rows23 = g * (L * 23) + lane23
        x = xv[pl.ds(g * L, L)]

        def expsum(off):
            w = [jnp.exp(plsc.load_gather(pv, [rows23 + (off + j)]))
                 for j in range(K)]
            return w, _prefix8(w)

        wx, cx = expsum(0)
        wy, cy = expsum(K)
        r = 1.0 / (cx[-1] * cy[-1])
        scale_x = (A_SC * r) * cy[-1]
        scale_y = (A_SC * r) * cx[-1]

        def positions(c, half_w0, scale, out_row):
            pos = []
            for j in range(K):
                pj = (c[j] - half_w0) * scale + (BETA * (j + 0.5) - B)
                posv[pl.ds(tbl + (out_row + j) * L, L)] = pj
                pos.append(pj)
            return pos

        xp = positions(cx, 0.5 * wx[0], scale_x, 2)
        positions(cy, 0.5 * wy[0], scale_y, 14)

        b0 = jnp.where(x > -B, 2, 1)
        b1 = (xp[0] < x).astype(jnp.int32) + (xp[1] < x).astype(jnp.int32)
        b2 = (xp[2] < x).astype(jnp.int32) + (xp[3] < x).astype(jnp.int32)
        b3 = (xp[4] < x).astype(jnp.int32) + (xp[5] < x).astype(jnp.int32)
        b4 = (xp[6] < x).astype(jnp.int32) + (xp[7] < x).astype(jnp.int32)
        b5 = (x > B).astype(jnp.int32)
        cnt = ((b0 + b1) + (b2 + b3)) + (b4 + b5)
        k = cnt - 1

        idx = tbl + k * L + lane
        xk = plsc.load_gather(posv, [idx])
        xk1 = plsc.load_gather(posv, [idx + L])
        yk = plsc.load_gather(posv, [idx + 12 * L])
        yk1 = plsc.load_gather(posv, [idx + 13 * L])

        jd = jnp.clip(k - 2, 0, 6) + 2 * K
        jd1 = jnp.clip(k - 1, 0, 6) + 2 * K
        pdk = plsc.load_gather(pv, [rows23 + jd])
        pdk1 = plsc.load_gather(pv, [rows23 + jd1])
        interior = jnp.logical_and(k > 1, k < 9)
        interior1 = jnp.logical_and(k > 0, k < 8)
        sp, sp1 = _softplus2(pdk, pdk1)
        dk = jnp.where(interior, sp + MIN_DERIVATIVE, 1.0)
        dk1 = jnp.where(interior1, sp1 + MIN_DERIVATIVE, 1.0)

        dx = xk1 - xk
        dy = yk1 - yk
        t = x - xk
        u = t * (dx - t)
        num = dy * (dy * (t * t) + (dk * dx) * u)
        den = dy * (dx * dx) + ((dk + dk1) * dx - 2.0 * dy) * u
        yv[pl.ds(g * L, L)] = yk + num / den

    def chunk_body(ci, carry):
        base = wid * per_w + ci * CH
        pltpu.sync_copy(x_hbm.at[pl.ds(base, CH)], xv)
        pltpu.sync_copy(p_hbm.at[pl.ds(base * 23, CH * 23)], pv)

        def grp2(h, carry2):
            group(2 * h, 0)
            group(2 * h + 1, TBL)
            return carry2

        lax.fori_loop(0, GRP // 2, grp2, 0)
        pltpu.sync_copy(yv, y_hbm.at[pl.ds(base, CH)])
        return carry

    lax.fori_loop(0, nch, chunk_body, 0)


@jax.jit
def kernel(x, params):
    f = pl.kernel(
        _sc_body,
        out_type=jax.ShapeDtypeStruct(x.shape, jnp.float32),
        mesh=plsc.VectorSubcoreMesh(core_axis_name="c", subcore_axis_name="s"),
        compiler_params=pltpu.CompilerParams(needs_layout_passes=False),
        scratch_types=[
            pltpu.VMEM((CH,), jnp.float32),
            pltpu.VMEM((CH * 23,), jnp.float32),
            pltpu.VMEM((CH,), jnp.float32),
            pltpu.VMEM((2 * TBL,), jnp.float32),
        ],
    )
    return f(x, params.reshape(-1))


# speedup vs baseline: 18.3357x; 3.1264x over previous
import jax
import jax.numpy as jnp
from jax import lax
from jax.experimental import pallas as pl
from jax.experimental.pallas import tpu as pltpu
from jax.experimental.pallas import tpu_sc as plsc

K = 8
B = 4.0
ADJ = 0.01
MIN_DERIVATIVE = 0.001
PAD_LO = -40000.0
PAD_HI = 40000.0

NC = 2
NS = 16
L = 16
NW = NC * NS

CH = 2048
GRP = CH // L
TBL = 24 * L

A_SC = 2.0 * B / (1.0 + ADJ)
BETA = 2.0 * B * (ADJ / K) / (1.0 + ADJ)


def _softplus2(t1, t2):
    e1 = jnp.exp(-jnp.abs(t1))
    e2 = jnp.exp(-jnp.abs(t2))
    a = e1 + 2.0
    b = e2 + 2.0
    r = 1.0 / (a * b)
    outs = []
    for t, e, other in ((t1, e1, b), (t2, e2, a)):
        z = e * other * r
        z2 = z * z
        p = 2.0 / 9.0 + z2 * (2.0 / 11.0)
        p = 2.0 / 7.0 + z2 * p
        p = 2.0 / 5.0 + z2 * p
        p = 2.0 / 3.0 + z2 * p
        p = 2.0 + z2 * p
        outs.append(jnp.maximum(t, 0.0) + z * p)
    return outs


def _prefix8(w):
    s01 = w[0] + w[1]
    s23 = w[2] + w[3]
    s45 = w[4] + w[5]
    s67 = w[6] + w[7]
    s03 = s01 + s23
    s47 = s45 + s67
    total = s03 + s47
    c = [w[0], s01, s01 + w[2], s03, s03 + w[4], s03 + s45,
         s03 + s45 + w[6], total]
    return c


def _sc_body(x_hbm, p_hbm, y_hbm, xv, pv, yv, posv):
    wid = lax.axis_index("s") * NC + lax.axis_index("c")
    per_w = x_hbm.shape[0] // NW
    nch = per_w // CH
    lane = lax.iota(jnp.int32, L)
    ones = jnp.ones((L,), jnp.float32)

    for r in range(2):
        o = r * TBL
        posv[pl.ds(o + 0 * L, L)] = PAD_LO * ones
        posv[pl.ds(o + 1 * L, L)] = -B * ones
        posv[pl.ds(o + 10 * L, L)] = B * ones
        posv[pl.ds(o + 11 * L, L)] = PAD_HI * ones
        posv[pl.ds(o + 12 * L, L)] = PAD_LO * ones
        posv[pl.ds(o + 13 * L, L)] = -B * ones
        posv[pl.ds(o + 22 * L, L)] = B * ones
        posv[pl.ds(o + 23 * L, L)] = PAD_HI * ones

    def group(g, tbl):
        eoff = g * L
        x = xv[pl.ds(eoff, L)]

        def expsum(off):
            w = [jnp.exp(pv[off + j, pl.ds(eoff, L)]) for j in range(K)]
            return w, _prefix8(w)

        wx, cx = expsum(0)
        wy, cy = expsum(K)
        r = 1.0 / (cx[-1] * cy[-1])
        scale_x = (A_SC * r) * cy[-1]
        scale_y = (A_SC * r) * cx[-1]

        def positions(c, half_w0, scale, out_row):
            pos = []
            for j in range(K):
                pj = (c[j] - half_w0) * scale + (BETA * (j + 0.5) - B)
                posv[pl.ds(tbl + (out_row + j) * L, L)] = pj
                pos.append(pj)
            return pos

        xp = positions(cx, 0.5 * wx[0], scale_x, 2)
        positions(cy, 0.5 * wy[0], scale_y, 14)

        b0 = jnp.where(x > -B, 2, 1)
        b1 = (xp[0] < x).astype(jnp.int32) + (xp[1] < x).astype(jnp.int32)
        b2 = (xp[2] < x).astype(jnp.int32) + (xp[3] < x).astype(jnp.int32)
        b3 = (xp[4] < x).astype(jnp.int32) + (xp[5] < x).astype(jnp.int32)
        b4 = (xp[6] < x).astype(jnp.int32) + (xp[7] < x).astype(jnp.int32)
        b5 = (x > B).astype(jnp.int32)
        cnt = ((b0 + b1) + (b2 + b3)) + (b4 + b5)
        k = cnt - 1

        idx = tbl + k * L + lane
        xk = plsc.load_gather(posv, [idx])
        xk1 = plsc.load_gather(posv, [idx + L])
        yk = plsc.load_gather(posv, [idx + 12 * L])
        yk1 = plsc.load_gather(posv, [idx + 13 * L])

        jd = jnp.clip(k - 2, 0, 6) + 2 * K
        jd1 = jnp.clip(k - 1, 0, 6) + 2 * K
        cols = eoff + lane
        pdk = plsc.load_gather(pv, [jd, cols])
        pdk1 = plsc.load_gather(pv, [jd1, cols])
        interior = jnp.logical_and(k > 1, k < 9)
        interior1 = jnp.logical_and(k > 0, k < 8)
        sp, sp1 = _softplus2(pdk, pdk1)
        dk = jnp.where(interior, sp + MIN_DERIVATIVE, 1.0)
        dk1 = jnp.where(interior1, sp1 + MIN_DERIVATIVE, 1.0)

        dx = xk1 - xk
        dy = yk1 - yk
        t = x - xk
        u = t * (dx - t)
        num = dy * (dy * (t * t) + (dk * dx) * u)
        den = dy * (dx * dx) + ((dk + dk1) * dx - 2.0 * dy) * u
        yv[pl.ds(g * L, L)] = yk + num / den

    def chunk_body(ci, carry):
        base = wid * per_w + ci * CH
        pltpu.sync_copy(x_hbm.at[pl.ds(base, CH)], xv)
        pltpu.sync_copy(p_hbm.at[:, pl.ds(base, CH)], pv)

        def grp2(h, carry2):
            group(2 * h, 0)
            group(2 * h + 1, TBL)
            return carry2

        lax.fori_loop(0, GRP // 2, grp2, 0)
        pltpu.sync_copy(yv, y_hbm.at[pl.ds(base, CH)])
        return carry

    lax.fori_loop(0, nch, chunk_body, 0)


@jax.jit
def kernel(x, params):
    f = pl.kernel(
        _sc_body,
        out_type=jax.ShapeDtypeStruct(x.shape, jnp.float32),
        mesh=plsc.VectorSubcoreMesh(core_axis_name="c", subcore_axis_name="s"),
        compiler_params=pltpu.CompilerParams(needs_layout_passes=False),
        scratch_types=[
            pltpu.VMEM((CH,), jnp.float32),
            pltpu.VMEM((23, CH), jnp.float32),
            pltpu.VMEM((CH,), jnp.float32),
            pltpu.VMEM((2 * TBL,), jnp.float32),
        ],
    )
    return f(x, params.T)


# speedup vs baseline: 22.5865x; 1.2318x over previous
import jax
import jax.numpy as jnp
from jax import lax
from jax.experimental import pallas as pl
from jax.experimental.pallas import tpu as pltpu
from jax.experimental.pallas import tpu_sc as plsc

K = 8
B = 4.0
ADJ = 0.01
MIN_DERIVATIVE = 0.001
PAD_LO = -40000.0
PAD_HI = 40000.0

NC = 2
NS = 16
L = 16
NW = NC * NS

CH = 2048
GRP = CH // L
TBL = 24 * L

A_SC = 2.0 * B / (1.0 + ADJ)
BETA = 2.0 * B * (ADJ / K) / (1.0 + ADJ)


def _softplus2(t1, t2):
    e1 = jnp.exp(-jnp.abs(t1))
    e2 = jnp.exp(-jnp.abs(t2))
    a = e1 + 2.0
    b = e2 + 2.0
    r = 1.0 / (a * b)
    outs = []
    for t, e, other in ((t1, e1, b), (t2, e2, a)):
        z = e * other * r
        z2 = z * z
        p = 2.0 / 9.0 + z2 * (2.0 / 11.0)
        p = 2.0 / 7.0 + z2 * p
        p = 2.0 / 5.0 + z2 * p
        p = 2.0 / 3.0 + z2 * p
        p = 2.0 + z2 * p
        outs.append(jnp.maximum(t, 0.0) + z * p)
    return outs


def _prefix8(w):
    s01 = w[0] + w[1]
    s23 = w[2] + w[3]
    s45 = w[4] + w[5]
    s67 = w[6] + w[7]
    s03 = s01 + s23
    s47 = s45 + s67
    total = s03 + s47
    c = [w[0], s01, s01 + w[2], s03, s03 + w[4], s03 + s45,
         s03 + s45 + w[6], total]
    return c


def _sc_body(x_hbm, p_hbm, y_hbm, xv0, pv0, yv0, xv1, pv1, yv1, posv,
             sin0, sin1, sout0, sout1):
    wid = lax.axis_index("s") * NC + lax.axis_index("c")
    per_w = x_hbm.shape[0] // NW
    nch = per_w // CH
    lane = lax.iota(jnp.int32, L)
    ones = jnp.ones((L,), jnp.float32)
    bufs = ((xv0, pv0, yv0, sin0, sout0), (xv1, pv1, yv1, sin1, sout1))

    for r in range(2):
        o = r * TBL
        posv[pl.ds(o + 0 * L, L)] = PAD_LO * ones
        posv[pl.ds(o + 1 * L, L)] = -B * ones
        posv[pl.ds(o + 10 * L, L)] = B * ones
        posv[pl.ds(o + 11 * L, L)] = PAD_HI * ones
        posv[pl.ds(o + 12 * L, L)] = PAD_LO * ones
        posv[pl.ds(o + 13 * L, L)] = -B * ones
        posv[pl.ds(o + 22 * L, L)] = B * ones
        posv[pl.ds(o + 23 * L, L)] = PAD_HI * ones

    def group(g, tbl, xv, pv, yv):
        eoff = g * L
        x = xv[pl.ds(eoff, L)]

        def expsum(off):
            w = [jnp.exp(pv[off + j, pl.ds(eoff, L)]) for j in range(K)]
            return w, _prefix8(w)

        wx, cx = expsum(0)
        wy, cy = expsum(K)
        r = 1.0 / (cx[-1] * cy[-1])
        scale_x = (A_SC * r) * cy[-1]
        scale_y = (A_SC * r) * cx[-1]

        def positions(c, half_w0, scale, out_row):
            pos = []
            for j in range(K):
                pj = (c[j] - half_w0) * scale + (BETA * (j + 0.5) - B)
                posv[pl.ds(tbl + (out_row + j) * L, L)] = pj
                pos.append(pj)
            return pos

        xp = positions(cx, 0.5 * wx[0], scale_x, 2)
        positions(cy, 0.5 * wy[0], scale_y, 14)

        b0 = jnp.where(x > -B, 2, 1)
        b1 = (xp[0] < x).astype(jnp.int32) + (xp[1] < x).astype(jnp.int32)
        b2 = (xp[2] < x).astype(jnp.int32) + (xp[3] < x).astype(jnp.int32)
        b3 = (xp[4] < x).astype(jnp.int32) + (xp[5] < x).astype(jnp.int32)
        b4 = (xp[6] < x).astype(jnp.int32) + (xp[7] < x).astype(jnp.int32)
        b5 = (x > B).astype(jnp.int32)
        cnt = ((b0 + b1) + (b2 + b3)) + (b4 + b5)
        k = cnt - 1

        idx = tbl + k * L + lane
        xk = plsc.load_gather(posv, [idx])
        xk1 = plsc.load_gather(posv, [idx + L])
        yk = plsc.load_gather(posv, [idx + 12 * L])
        yk1 = plsc.load_gather(posv, [idx + 13 * L])

        jd = jnp.clip(k - 2, 0, 6) + 2 * K
        jd1 = jnp.clip(k - 1, 0, 6) + 2 * K
        cols = eoff + lane
        pdk = plsc.load_gather(pv, [jd, cols])
        pdk1 = plsc.load_gather(pv, [jd1, cols])
        interior = jnp.logical_and(k > 1, k < 9)
        interior1 = jnp.logical_and(k > 0, k < 8)
        sp, sp1 = _softplus2(pdk, pdk1)
        dk = jnp.where(interior, sp + MIN_DERIVATIVE, 1.0)
        dk1 = jnp.where(interior1, sp1 + MIN_DERIVATIVE, 1.0)

        dx = xk1 - xk
        dy = yk1 - yk
        t = x - xk
        u = t * (dx - t)
        num = dy * (dy * (t * t) + (dk * dx) * u)
        den = dy * (dx * dx) + ((dk + dk1) * dx - 2.0 * dy) * u
        yv[pl.ds(g * L, L)] = yk + num / den

    base_w = wid * per_w

    def start_in(ci, b):
        xv, pv, _, sin, _ = bufs[b]
        base = base_w + ci * CH
        pltpu.async_copy(x_hbm.at[pl.ds(base, CH)], xv, sin)
        pltpu.async_copy(p_hbm.at[:, pl.ds(base, CH)], pv, sin)

    def wait_in(b):
        xv, pv, _, sin, _ = bufs[b]
        pltpu.make_async_copy(x_hbm.at[pl.ds(0, CH)], xv, sin).wait()
        pltpu.make_async_copy(p_hbm.at[:, pl.ds(0, CH)], pv, sin).wait()

    def start_out(ci, b):
        _, _, yv, _, sout = bufs[b]
        pltpu.async_copy(yv, y_hbm.at[pl.ds(base_w + ci * CH, CH)], sout)

    def wait_out(b):
        _, _, yv, _, sout = bufs[b]
        pltpu.make_async_copy(yv, y_hbm.at[pl.ds(0, CH)], sout).wait()

    def compute(xv, pv, yv):
        def grp2(h, carry2):
            group(2 * h, 0, xv, pv, yv)
            group(2 * h + 1, TBL, xv, pv, yv)
            return carry2
        lax.fori_loop(0, GRP // 2, grp2, 0)

    start_in(0, 0)
    start_in(1, 1)

    def chunk_pair(i, carry):
        for b in range(2):
            ci = 2 * i + b
            xv, pv, yv, _, _ = bufs[b]
            wait_in(b)

            @pl.when(i > 0)
            def _():
                wait_out(b)

            compute(xv, pv, yv)
            start_out(ci, b)

            @pl.when(i < (nch // 2 - 1))
            def _():
                start_in(ci + 2, b)

        return carry

    lax.fori_loop(0, nch // 2, chunk_pair, 0)
    wait_out(0)
    wait_out(1)


@jax.jit
def kernel(x, params):
    f = pl.kernel(
        _sc_body,
        out_type=jax.ShapeDtypeStruct(x.shape, jnp.float32),
        mesh=plsc.VectorSubcoreMesh(core_axis_name="c", subcore_axis_name="s"),
        compiler_params=pltpu.CompilerParams(needs_layout_passes=False),
        scratch_types=[
            pltpu.VMEM((CH,), jnp.float32),
            pltpu.VMEM((23, CH), jnp.float32),
            pltpu.VMEM((CH,), jnp.float32),
            pltpu.VMEM((CH,), jnp.float32),
            pltpu.VMEM((23, CH), jnp.float32),
            pltpu.VMEM((CH,), jnp.float32),
            pltpu.VMEM((2 * TBL,), jnp.float32),
            pltpu.SemaphoreType.DMA,
            pltpu.SemaphoreType.DMA,
            pltpu.SemaphoreType.DMA,
            pltpu.SemaphoreType.DMA,
        ],
    )
    return f(x, params.T)


# speedup vs baseline: 40.4293x; 1.7900x over previous
import jax
import jax.numpy as jnp
from jax import lax
from jax.experimental import pallas as pl
from jax.experimental.pallas import tpu as pltpu
from jax.experimental.pallas import tpu_sc as plsc

K = 8
B = 4.0
ADJ = 0.01
MIN_DERIVATIVE = 0.001
PAD_LO = -40000.0
PAD_HI = 40000.0

NC = 2
NS = 16
L = 16
NW = NC * NS

CH = 2048
GRP = CH // L
TBL = 24 * L

A_SC = 2.0 * B / (1.0 + ADJ)
BETA = 2.0 * B * (ADJ / K) / (1.0 + ADJ)


def _softplus2(t1, t2):
    e1 = jnp.exp(-jnp.abs(t1))
    e2 = jnp.exp(-jnp.abs(t2))
    a = e1 + 2.0
    b = e2 + 2.0
    r = 1.0 / (a * b)
    outs = []
    for t, e, other in ((t1, e1, b), (t2, e2, a)):
        z = e * other * r
        z2 = z * z
        p = 2.0 / 9.0 + z2 * (2.0 / 11.0)
        p = 2.0 / 7.0 + z2 * p
        p = 2.0 / 5.0 + z2 * p
        p = 2.0 / 3.0 + z2 * p
        p = 2.0 + z2 * p
        outs.append(jnp.maximum(t, 0.0) + z * p)
    return outs


def _prefix8(w):
    s01 = w[0] + w[1]
    s23 = w[2] + w[3]
    s45 = w[4] + w[5]
    s67 = w[6] + w[7]
    s03 = s01 + s23
    s47 = s45 + s67
    total = s03 + s47
    c = [w[0], s01, s01 + w[2], s03, s03 + w[4], s03 + s45,
         s03 + s45 + w[6], total]
    return c


def _sc_body(x_hbm, p_hbm, y_hbm, xv0, pv0, yv0, xv1, pv1, yv1,
             sin0, sin1, sout0, sout1):
    wid = lax.axis_index("s") * NC + lax.axis_index("c")
    per_w = x_hbm.shape[0] // NW
    nch = per_w // CH
    bufs = ((xv0, pv0, yv0, sin0, sout0), (xv1, pv1, yv1, sin1, sout1))

    def group(g, xv, pv, yv):
        eoff = g * L
        x = xv[pl.ds(eoff, L)]

        def expsum(off):
            w = [jnp.exp(pv[off + j, pl.ds(eoff, L)]) for j in range(K)]
            return w, _prefix8(w)

        wx, cx = expsum(0)
        wy, cy = expsum(K)
        r = 1.0 / (cx[-1] * cy[-1])
        scale_x = (A_SC * r) * cy[-1]
        scale_y = (A_SC * r) * cx[-1]

        def positions(c, half_w0, scale):
            return [(c[j] - half_w0) * scale + (BETA * (j + 0.5) - B)
                    for j in range(K)]

        xp = positions(cx, 0.5 * wx[0], scale_x)
        yp = positions(cy, 0.5 * wy[0], scale_y)

        m = [xp[j] < x for j in range(K)]
        mlo = x > -B
        mhi = x > B

        def sel_lo(p):
            v = jnp.where(mlo, -B, PAD_LO)
            for j in range(K):
                v = jnp.where(m[j], p[j], v)
            return jnp.where(mhi, B, v)

        def sel_hi(p):
            v = jnp.where(mhi, PAD_HI, B)
            for j in reversed(range(K)):
                v = jnp.where(m[j], v, p[j])
            return jnp.where(mlo, v, -B)

        xk = sel_lo(xp)
        xk1 = sel_hi(xp)
        yk = sel_lo(yp)
        yk1 = sel_hi(yp)

        rows = [pv[2 * K + i, pl.ds(eoff, L)] for i in range(K - 1)]
        pdk = rows[0]
        pdk1 = rows[0]
        for i in range(1, K - 1):
            pdk = jnp.where(m[i], rows[i], pdk)
            pdk1 = jnp.where(m[i - 1], rows[i], pdk1)
        interior = jnp.logical_and(mlo, jnp.logical_and(
            m[0], jnp.logical_not(m[K - 1])))
        interior1 = jnp.logical_and(mlo, jnp.logical_not(m[K - 2]))
        sp, sp1 = _softplus2(pdk, pdk1)
        dk = jnp.where(interior, sp + MIN_DERIVATIVE, 1.0)
        dk1 = jnp.where(interior1, sp1 + MIN_DERIVATIVE, 1.0)

        dx = xk1 - xk
        dy = yk1 - yk
        t = x - xk
        u = t * (dx - t)
        num = dy * (dy * (t * t) + (dk * dx) * u)
        den = dy * (dx * dx) + ((dk + dk1) * dx - 2.0 * dy) * u
        yv[pl.ds(g * L, L)] = yk + num / den

    base_w = wid * per_w

    def start_in(ci, b):
        xv, pv, _, sin, _ = bufs[b]
        base = base_w + ci * CH
        pltpu.async_copy(x_hbm.at[pl.ds(base, CH)], xv, sin)
        pltpu.async_copy(p_hbm.at[:, pl.ds(base, CH)], pv, sin)

    def wait_in(b):
        xv, pv, _, sin, _ = bufs[b]
        pltpu.make_async_copy(x_hbm.at[pl.ds(0, CH)], xv, sin).wait()
        pltpu.make_async_copy(p_hbm.at[:, pl.ds(0, CH)], pv, sin).wait()

    def start_out(ci, b):
        _, _, yv, _, sout = bufs[b]
        pltpu.async_copy(yv, y_hbm.at[pl.ds(base_w + ci * CH, CH)], sout)

    def wait_out(b):
        _, _, yv, _, sout = bufs[b]
        pltpu.make_async_copy(yv, y_hbm.at[pl.ds(0, CH)], sout).wait()

    def compute(xv, pv, yv):
        @plsc.parallel_loop(0, GRP, 1, unroll=2)
        def _(g):
            group(g, xv, pv, yv)

    start_in(0, 0)
    start_in(1, 1)

    def chunk_pair(i, carry):
        for b in range(2):
            ci = 2 * i + b
            xv, pv, yv, _, _ = bufs[b]
            wait_in(b)

            @pl.when(i > 0)
            def _():
                wait_out(b)

            compute(xv, pv, yv)
            start_out(ci, b)

            @pl.when(i < (nch // 2 - 1))
            def _():
                start_in(ci + 2, b)

        return carry

    lax.fori_loop(0, nch // 2, chunk_pair, 0)
    wait_out(0)
    wait_out(1)


@jax.jit
def kernel(x, params):
    f = pl.kernel(
        _sc_body,
        out_type=jax.ShapeDtypeStruct(x.shape, jnp.float32),
        mesh=plsc.VectorSubcoreMesh(core_axis_name="c", subcore_axis_name="s"),
        compiler_params=pltpu.CompilerParams(needs_layout_passes=False),
        scratch_types=[
            pltpu.VMEM((CH,), jnp.float32),
            pltpu.VMEM((23, CH), jnp.float32),
            pltpu.VMEM((CH,), jnp.float32),
            pltpu.VMEM((CH,), jnp.float32),
            pltpu.VMEM((23, CH), jnp.float32),
            pltpu.VMEM((CH,), jnp.float32),
            pltpu.SemaphoreType.DMA,
            pltpu.SemaphoreType.DMA,
            pltpu.SemaphoreType.DMA,
            pltpu.SemaphoreType.DMA,
        ],
    )
    return f(x, params.T)
